# baseline (speedup 1.0000x reference)
import functools

import jax
import jax.numpy as jnp
from jax import lax
from jax.experimental import pallas as pl
from jax.experimental.pallas import tpu as pltpu
from jax.experimental.pallas import tpu_sc as plsc

BATCH = 16384
D0 = 32
D1 = 32
D2 = 64
HIDDEN = 128
V = 100000

_NC = 2
_NS = 16
_NW = _NC * _NS
_RPW = (D0 + D1 + D2) // _NW
_SA = 50048
_SB = V - _SA
_Q = 4096
_NQ = BATCH // _Q
_L = 16


def _gather_body(i0, i1, i2, t0T, t1T, t2T, e0T, e1T, e2T,
                 bufA, bufB, idx_r, out_f, semA, semB, semI, semO):
    wid = lax.axis_index("s") * _NC + lax.axis_index("c")

    def do_table(tbl, ev, ihbm, base):
        dA = pltpu.async_copy(tbl.at[base, pl.ds(0, _SA)], bufA, semA)
        dB = pltpu.async_copy(tbl.at[base, pl.ds(_SA, _SB)], bufB, semB)
        di = pltpu.async_copy(ihbm.at[pl.ds(0, _Q)], idx_r.at[0], semI)
        cur = 0
        dO = None
        for k in range(_RPW):
            c = base + k
            dA.wait()
            if dO is not None:
                dO.wait()
            for q in range(_NQ):
                di.wait()
                di = pltpu.async_copy(
                    ihbm.at[pl.ds(((q + 1) % _NQ) * _Q, _Q)],
                    idx_r.at[1 - cur], semI)
                s = cur

                def pass_a(j, carry, q=q, s=s):
                    iv = idx_r[s, pl.ds(j * _L, _L)]
                    ivc = jnp.minimum(iv, _SA - 1)
                    out_f[pl.ds(q * _Q + j * _L, _L)] = plsc.load_gather(
                        bufA, [ivc])
                    return carry

                lax.fori_loop(0, _Q // _L, pass_a, 0, unroll=8)
                cur = 1 - cur
            if k < _RPW - 1:
                dA = pltpu.async_copy(tbl.at[c + 1, pl.ds(0, _SA)], bufA, semA)
            dB.wait()
            for q in range(_NQ):
                di.wait()
                di = pltpu.async_copy(
                    ihbm.at[pl.ds(((q + 1) % _NQ) * _Q, _Q)],
                    idx_r.at[1 - cur], semI)
                s = cur

                def pass_b(j, carry, q=q, s=s):
                    iv = idx_r[s, pl.ds(j * _L, _L)]
                    ivb = jnp.maximum(iv - _SA, 0)
                    g = plsc.load_gather(bufB, [ivb])
                    sl = pl.ds(q * _Q + j * _L, _L)
                    out_f[sl] = jnp.where(iv >= _SA, g, out_f[sl])
                    return carry

                lax.fori_loop(0, _Q // _L, pass_b, 0, unroll=8)
                cur = 1 - cur
            if k < _RPW - 1:
                dB = pltpu.async_copy(tbl.at[c + 1, pl.ds(_SA, _SB)], bufB, semB)
            dO = pltpu.async_copy(out_f, ev.at[c], semO)
        dO.wait()
        di.wait()

    @pl.when(wid < 8)
    def _():
        do_table(t0T, e0T, i0, wid * _RPW)

    @pl.when((wid >= 8) & (wid < 16))
    def _():
        do_table(t1T, e1T, i1, (wid - 8) * _RPW)

    @pl.when(wid >= 16)
    def _():
        do_table(t2T, e2T, i2, (wid - 16) * _RPW)


@functools.cache
def _make_gather():
    return pl.kernel(
        _gather_body,
        out_type=[
            jax.ShapeDtypeStruct((D0, BATCH), jnp.float32),
            jax.ShapeDtypeStruct((D1, BATCH), jnp.float32),
            jax.ShapeDtypeStruct((D2, BATCH), jnp.float32),
        ],
        mesh=plsc.VectorSubcoreMesh(core_axis_name="c", subcore_axis_name="s"),
        scratch_types=[
            pltpu.VMEM((_SA,), jnp.float32),
            pltpu.VMEM((_SB,), jnp.float32),
            pltpu.VMEM((2, _Q), jnp.int32),
            pltpu.VMEM((BATCH,), jnp.float32),
            pltpu.SemaphoreType.DMA,
            pltpu.SemaphoreType.DMA,
            pltpu.SemaphoreType.DMA,
            pltpu.SemaphoreType.DMA,
        ],
        compiler_params=pltpu.CompilerParams(needs_layout_passes=False),
    )


_MM_COLS = 1024


def _mm_body(e0_ref, e1_ref, e2_ref, w0_ref, w1_ref, w2_ref, b_ref, o_ref):
    dn = (((0,), (0,)), ((), ()))
    acc = lax.dot_general(e0_ref[...], w0_ref[...], dn,
                          preferred_element_type=jnp.float32)
    acc += lax.dot_general(e1_ref[...], w1_ref[...], dn,
                           preferred_element_type=jnp.float32)
    acc += lax.dot_general(e2_ref[...], w2_ref[...], dn,
                           preferred_element_type=jnp.float32)
    o_ref[...] = acc + b_ref[...]


_matmul = pl.pallas_call(
    _mm_body,
    grid=(BATCH // _MM_COLS,),
    in_specs=[
        pl.BlockSpec((D0, _MM_COLS), lambda i: (0, i)),
        pl.BlockSpec((D1, _MM_COLS), lambda i: (0, i)),
        pl.BlockSpec((D2, _MM_COLS), lambda i: (0, i)),
        pl.BlockSpec((D0, HIDDEN), lambda i: (0, 0)),
        pl.BlockSpec((D1, HIDDEN), lambda i: (0, 0)),
        pl.BlockSpec((D2, HIDDEN), lambda i: (0, 0)),
        pl.BlockSpec((1, HIDDEN), lambda i: (0, 0)),
    ],
    out_specs=pl.BlockSpec((_MM_COLS, HIDDEN), lambda i: (i, 0)),
    out_shape=jax.ShapeDtypeStruct((BATCH, HIDDEN), jnp.float32),
    compiler_params=pltpu.CompilerParams(fuse_transposed_lhs_in_matmul=True),
)


@jax.jit
def kernel(segment_features, lane_table, type_table, length_table, W, b):
    idx = segment_features.astype(jnp.int32)
    e0T, e1T, e2T = _make_gather()(
        idx[:, 0], idx[:, 1], idx[:, 2],
        lane_table.T, type_table.T, length_table.T)
    Wt = W.T
    return _matmul(e0T, e1T, e2T,
                   Wt[:D0], Wt[D0:D0 + D1], Wt[D0 + D1:],
                   b.reshape(1, HIDDEN))


# speedup vs baseline: 1.9652x; 1.9652x over previous
import functools

import jax
import jax.numpy as jnp
from jax import lax
from jax.experimental import pallas as pl
from jax.experimental.pallas import tpu as pltpu
from jax.experimental.pallas import tpu_sc as plsc

BATCH = 16384
D0 = 32
D1 = 32
D2 = 64
HIDDEN = 128
V = 100000

_NC = 2
_NS = 16
_NW = _NC * _NS
_RPW = (D0 + D1 + D2) // _NW
_OCHUNK = 4096
_L = 16


def _gather_body(i0, i1, i2, t0T, t1T, t2T, e0T, e1T, e2T,
                 idx_v, row_v, out_v):
    wid = lax.axis_index("s") * _NC + lax.axis_index("c")

    def do_table(tbl, ev, idx_hbm, base):
        pltpu.sync_copy(idx_hbm, idx_v)
        for k in range(_RPW):
            c = base + k
            pltpu.sync_copy(tbl.at[c], row_v)
            for h in range(BATCH // _OCHUNK):
                def gbody(j, carry):
                    base = j * (_L * 8)
                    ivs = [idx_v[pl.ds(h * _OCHUNK + base + t * _L, _L)]
                           for t in range(8)]
                    gs = [plsc.load_gather(row_v, [iv]) for iv in ivs]
                    for t in range(8):
                        out_v[pl.ds(base + t * _L, _L)] = gs[t]
                    return carry
                lax.fori_loop(0, _OCHUNK // (_L * 8), gbody, 0, unroll=2)
                pltpu.sync_copy(out_v, ev.at[c, pl.ds(h * _OCHUNK, _OCHUNK)])

    @pl.when(wid < 8)
    def _():
        do_table(t0T, e0T, i0, wid * _RPW)

    @pl.when((wid >= 8) & (wid < 16))
    def _():
        do_table(t1T, e1T, i1, (wid - 8) * _RPW)

    @pl.when(wid >= 16)
    def _():
        do_table(t2T, e2T, i2, (wid - 16) * _RPW)


@functools.cache
def _make_gather():
    return pl.kernel(
        _gather_body,
        out_type=[
            jax.ShapeDtypeStruct((D0, BATCH), jnp.float32),
            jax.ShapeDtypeStruct((D1, BATCH), jnp.float32),
            jax.ShapeDtypeStruct((D2, BATCH), jnp.float32),
        ],
        mesh=plsc.VectorSubcoreMesh(core_axis_name="c", subcore_axis_name="s"),
        scratch_types=[
            pltpu.VMEM((BATCH,), jnp.int32),
            pltpu.VMEM((V,), jnp.float32),
            pltpu.VMEM((_OCHUNK,), jnp.float32),
        ],
        compiler_params=pltpu.CompilerParams(needs_layout_passes=False),
    )


_MM_COLS = 1024


def _mm_body(e0_ref, e1_ref, e2_ref, w0_ref, w1_ref, w2_ref, b_ref, o_ref):
    dn = (((0,), (0,)), ((), ()))
    acc = lax.dot_general(e0_ref[...], w0_ref[...], dn,
                          preferred_element_type=jnp.float32)
    acc += lax.dot_general(e1_ref[...], w1_ref[...], dn,
                           preferred_element_type=jnp.float32)
    acc += lax.dot_general(e2_ref[...], w2_ref[...], dn,
                           preferred_element_type=jnp.float32)
    o_ref[...] = acc + b_ref[...]


_matmul = pl.pallas_call(
    _mm_body,
    grid=(BATCH // _MM_COLS,),
    in_specs=[
        pl.BlockSpec((D0, _MM_COLS), lambda i: (0, i)),
        pl.BlockSpec((D1, _MM_COLS), lambda i: (0, i)),
        pl.BlockSpec((D2, _MM_COLS), lambda i: (0, i)),
        pl.BlockSpec((D0, HIDDEN), lambda i: (0, 0)),
        pl.BlockSpec((D1, HIDDEN), lambda i: (0, 0)),
        pl.BlockSpec((D2, HIDDEN), lambda i: (0, 0)),
        pl.BlockSpec((1, HIDDEN), lambda i: (0, 0)),
    ],
    out_specs=pl.BlockSpec((_MM_COLS, HIDDEN), lambda i: (i, 0)),
    out_shape=jax.ShapeDtypeStruct((BATCH, HIDDEN), jnp.float32),
)


@jax.jit
def kernel(segment_features, lane_table, type_table, length_table, W, b):
    idx = segment_features.astype(jnp.int32)
    e0T, e1T, e2T = _make_gather()(
        idx[:, 0], idx[:, 1], idx[:, 2],
        lane_table.T, type_table.T, length_table.T)
    Wt = W.T
    return _matmul(e0T, e1T, e2T,
                   Wt[:D0], Wt[D0:D0 + D1], Wt[D0 + D1:],
                   b.reshape(1, HIDDEN))


# speedup vs baseline: 2.1020x; 1.0696x over previous
import functools

import jax
import jax.numpy as jnp
from jax import lax
from jax.experimental import pallas as pl
from jax.experimental.pallas import tpu as pltpu
from jax.experimental.pallas import tpu_sc as plsc

BATCH = 16384
D0 = 32
D1 = 32
D2 = 64
HIDDEN = 128
V = 100000

_NC = 2
_NS = 16
_NW = _NC * _NS
_RPW = (D0 + D1 + D2) // _NW
_OCHUNK = 4096
_L = 16


def _gather_body(i0, i1, i2, t0T, t1T, t2T, e0T, e1T, e2T,
                 idx_v, row_v, out_r, semI, semR, semO):
    wid = lax.axis_index("s") * _NC + lax.axis_index("c")

    def do_table(tbl, ev, idx_hbm, base):
        dI = pltpu.async_copy(idx_hbm, idx_v, semI)
        dR = pltpu.async_copy(tbl.at[base], row_v, semR)
        dI.wait()
        pend = [None, None]
        for k in range(_RPW):
            c = base + k
            dR.wait()
            for h in range(BATCH // _OCHUNK):
                s = h % 2
                if pend[s] is not None:
                    pend[s].wait()
                    pend[s] = None

                def gbody(j, carry, h=h, s=s):
                    off = j * (_L * 8)
                    ivs = [idx_v[pl.ds(h * _OCHUNK + off + t * _L, _L)]
                           for t in range(8)]
                    gs = [plsc.load_gather(row_v, [iv]) for iv in ivs]
                    for t in range(8):
                        out_r[s, pl.ds(off + t * _L, _L)] = gs[t]
                    return carry
                lax.fori_loop(0, _OCHUNK // (_L * 8), gbody, 0, unroll=2)
                pend[s] = pltpu.async_copy(
                    out_r.at[s], ev.at[c, pl.ds(h * _OCHUNK, _OCHUNK)], semO)
            if k < _RPW - 1:
                dR = pltpu.async_copy(tbl.at[c + 1], row_v, semR)
        for s in range(2):
            if pend[s] is not None:
                pend[s].wait()

    @pl.when(wid < 8)
    def _():
        do_table(t0T, e0T, i0, wid * _RPW)

    @pl.when((wid >= 8) & (wid < 16))
    def _():
        do_table(t1T, e1T, i1, (wid - 8) * _RPW)

    @pl.when(wid >= 16)
    def _():
        do_table(t2T, e2T, i2, (wid - 16) * _RPW)


@functools.cache
def _make_gather():
    return pl.kernel(
        _gather_body,
        out_type=[
            jax.ShapeDtypeStruct((D0, BATCH), jnp.float32),
            jax.ShapeDtypeStruct((D1, BATCH), jnp.float32),
            jax.ShapeDtypeStruct((D2, BATCH), jnp.float32),
        ],
        mesh=plsc.VectorSubcoreMesh(core_axis_name="c", subcore_axis_name="s"),
        scratch_types=[
            pltpu.VMEM((BATCH,), jnp.int32),
            pltpu.VMEM((V,), jnp.float32),
            pltpu.VMEM((2, _OCHUNK), jnp.float32),
            pltpu.SemaphoreType.DMA,
            pltpu.SemaphoreType.DMA,
            pltpu.SemaphoreType.DMA,
        ],
        compiler_params=pltpu.CompilerParams(needs_layout_passes=False),
    )


_MM_COLS = 2048


def _mm_body(e0_ref, e1_ref, e2_ref, w0_ref, w1_ref, w2_ref, b_ref, o_ref):
    dn = (((0,), (0,)), ((), ()))
    acc = lax.dot_general(e0_ref[...], w0_ref[...], dn,
                          preferred_element_type=jnp.float32)
    acc += lax.dot_general(e1_ref[...], w1_ref[...], dn,
                           preferred_element_type=jnp.float32)
    acc += lax.dot_general(e2_ref[...], w2_ref[...], dn,
                           preferred_element_type=jnp.float32)
    o_ref[...] = acc + b_ref[...]


_matmul = pl.pallas_call(
    _mm_body,
    grid=(BATCH // _MM_COLS,),
    in_specs=[
        pl.BlockSpec((D0, _MM_COLS), lambda i: (0, i)),
        pl.BlockSpec((D1, _MM_COLS), lambda i: (0, i)),
        pl.BlockSpec((D2, _MM_COLS), lambda i: (0, i)),
        pl.BlockSpec((D0, HIDDEN), lambda i: (0, 0)),
        pl.BlockSpec((D1, HIDDEN), lambda i: (0, 0)),
        pl.BlockSpec((D2, HIDDEN), lambda i: (0, 0)),
        pl.BlockSpec((1, HIDDEN), lambda i: (0, 0)),
    ],
    out_specs=pl.BlockSpec((_MM_COLS, HIDDEN), lambda i: (i, 0)),
    out_shape=jax.ShapeDtypeStruct((BATCH, HIDDEN), jnp.float32),
    compiler_params=pltpu.CompilerParams(fuse_transposed_lhs_in_matmul=True),
)


@jax.jit
def kernel(segment_features, lane_table, type_table, length_table, W, b):
    idx = segment_features.astype(jnp.int32)
    e0T, e1T, e2T = _make_gather()(
        idx[:, 0], idx[:, 1], idx[:, 2],
        lane_table.T, type_table.T, length_table.T)
    Wt = W.T
    return _matmul(e0T, e1T, e2T,
                   Wt[:D0], Wt[D0:D0 + D1], Wt[D0 + D1:],
                   b.reshape(1, HIDDEN))


# speedup vs baseline: 2.1462x; 1.0210x over previous
import functools

import jax
import jax.numpy as jnp
from jax import lax
from jax.experimental import pallas as pl
from jax.experimental.pallas import tpu as pltpu
from jax.experimental.pallas import tpu_sc as plsc

BATCH = 16384
D0 = 32
D1 = 32
D2 = 64
HIDDEN = 128
V = 100000

_NC = 2
_NS = 16
_NW = _NC * _NS
_RPW = (D0 + D1 + D2) // _NW
_OCHUNK = 4096
_L = 16


def _gather_body(i0, i1, i2, t0T, t1T, t2T, e0T, e1T, e2T,
                 idx_v, row_v, out_r, semI, semR, semO):
    wid = lax.axis_index("s") * _NC + lax.axis_index("c")

    @pl.when((wid & 1) == 1)
    def _():
        t = lax.fori_loop(0, 1500, lambda i, a: a + 1, 0)
        out_r[0, pl.ds(0, _L)] = jnp.full((_L,), t, jnp.float32)

    def do_table(tbl, ev, idx_hbm, base):
        dI = pltpu.async_copy(idx_hbm, idx_v, semI)
        dR = pltpu.async_copy(tbl.at[base], row_v, semR)
        dI.wait()
        pend = [None, None]
        for k in range(_RPW):
            c = base + k
            dR.wait()
            for h in range(BATCH // _OCHUNK):
                s = h % 2
                if pend[s] is not None:
                    pend[s].wait()
                    pend[s] = None

                def gbody(j, carry, h=h, s=s):
                    off = j * (_L * 8)
                    ivs = [idx_v[pl.ds(h * _OCHUNK + off + t * _L, _L)]
                           for t in range(8)]
                    gs = [plsc.load_gather(row_v, [iv]) for iv in ivs]
                    for t in range(8):
                        out_r[s, pl.ds(off + t * _L, _L)] = gs[t]
                    return carry
                lax.fori_loop(0, _OCHUNK // (_L * 8), gbody, 0, unroll=2)
                pend[s] = pltpu.async_copy(
                    out_r.at[s], ev.at[c, pl.ds(h * _OCHUNK, _OCHUNK)], semO)
            if k < _RPW - 1:
                dR = pltpu.async_copy(tbl.at[c + 1], row_v, semR)
        for s in range(2):
            if pend[s] is not None:
                pend[s].wait()

    @pl.when(wid < 8)
    def _():
        do_table(t0T, e0T, i0, wid * _RPW)

    @pl.when((wid >= 8) & (wid < 16))
    def _():
        do_table(t1T, e1T, i1, (wid - 8) * _RPW)

    @pl.when(wid >= 16)
    def _():
        do_table(t2T, e2T, i2, (wid - 16) * _RPW)


@functools.cache
def _make_gather():
    return pl.kernel(
        _gather_body,
        out_type=[
            jax.ShapeDtypeStruct((D0, BATCH), jnp.float32),
            jax.ShapeDtypeStruct((D1, BATCH), jnp.float32),
            jax.ShapeDtypeStruct((D2, BATCH), jnp.float32),
        ],
        mesh=plsc.VectorSubcoreMesh(core_axis_name="c", subcore_axis_name="s"),
        scratch_types=[
            pltpu.VMEM((BATCH,), jnp.int32),
            pltpu.VMEM((V,), jnp.float32),
            pltpu.VMEM((2, _OCHUNK), jnp.float32),
            pltpu.SemaphoreType.DMA,
            pltpu.SemaphoreType.DMA,
            pltpu.SemaphoreType.DMA,
        ],
        compiler_params=pltpu.CompilerParams(needs_layout_passes=False),
    )


_MM_COLS = 4096


def _mm_body(e0_ref, e1_ref, e2_ref, w0_ref, w1_ref, w2_ref, b_ref, o_ref):
    dn = (((0,), (0,)), ((), ()))
    acc = lax.dot_general(e0_ref[...], w0_ref[...], dn,
                          preferred_element_type=jnp.float32)
    acc += lax.dot_general(e1_ref[...], w1_ref[...], dn,
                           preferred_element_type=jnp.float32)
    acc += lax.dot_general(e2_ref[...], w2_ref[...], dn,
                           preferred_element_type=jnp.float32)
    o_ref[...] = acc + b_ref[...]


_matmul = pl.pallas_call(
    _mm_body,
    grid=(BATCH // _MM_COLS,),
    in_specs=[
        pl.BlockSpec((D0, _MM_COLS), lambda i: (0, i)),
        pl.BlockSpec((D1, _MM_COLS), lambda i: (0, i)),
        pl.BlockSpec((D2, _MM_COLS), lambda i: (0, i)),
        pl.BlockSpec((D0, HIDDEN), lambda i: (0, 0)),
        pl.BlockSpec((D1, HIDDEN), lambda i: (0, 0)),
        pl.BlockSpec((D2, HIDDEN), lambda i: (0, 0)),
        pl.BlockSpec((1, HIDDEN), lambda i: (0, 0)),
    ],
    out_specs=pl.BlockSpec((_MM_COLS, HIDDEN), lambda i: (i, 0)),
    out_shape=jax.ShapeDtypeStruct((BATCH, HIDDEN), jnp.float32),
    compiler_params=pltpu.CompilerParams(fuse_transposed_lhs_in_matmul=True),
)


@jax.jit
def kernel(segment_features, lane_table, type_table, length_table, W, b):
    idx = segment_features.astype(jnp.int32)
    e0T, e1T, e2T = _make_gather()(
        idx[:, 0], idx[:, 1], idx[:, 2],
        lane_table.T, type_table.T, length_table.T)
    Wt = W.T
    return _matmul(e0T, e1T, e2T,
                   Wt[:D0], Wt[D0:D0 + D1], Wt[D0 + D1:],
                   b.reshape(1, HIDDEN))


# speedup vs baseline: 2.2476x; 1.0472x over previous
import functools

import jax
import jax.numpy as jnp
from jax import lax
from jax.experimental import pallas as pl
from jax.experimental.pallas import tpu as pltpu
from jax.experimental.pallas import tpu_sc as plsc

BATCH = 16384
D0 = 32
D1 = 32
D2 = 64
DTOT = D0 + D1 + D2
HIDDEN = 128
V = 100000

_NC = 2
_NS = 16
_NW = _NC * _NS
_RPW = DTOT // _NW
_OCHUNK = 4096
_L = 16


def _gather_body(i0, i1, i2, t0T, t1T, t2T, eT,
                 idx_v, row_v, out_r, semI, semR, semO):
    wid = lax.axis_index("s") * _NC + lax.axis_index("c")

    @pl.when((wid & 3) > 0)
    def _():
        t = lax.fori_loop(0, 750 * (wid & 3), lambda i, a: a + 1, 0)
        out_r[0, pl.ds(0, _L)] = jnp.full((_L,), t, jnp.float32)

    def do_table(tbl, ev_base, idx_hbm, base):
        dI = pltpu.async_copy(idx_hbm, idx_v, semI)
        dR = pltpu.async_copy(tbl.at[base], row_v, semR)
        dI.wait()
        pend = [None, None]
        for k in range(_RPW):
            c = base + k
            dR.wait()
            for h in range(BATCH // _OCHUNK):
                s = h % 2
                if pend[s] is not None:
                    pend[s].wait()
                    pend[s] = None

                def gbody(j, carry, h=h, s=s):
                    off = j * (_L * 8)
                    ivs = [idx_v[pl.ds(h * _OCHUNK + off + t * _L, _L)]
                           for t in range(8)]
                    gs = [plsc.load_gather(row_v, [iv]) for iv in ivs]
                    for t in range(8):
                        out_r[s, pl.ds(off + t * _L, _L)] = gs[t]
                    return carry
                lax.fori_loop(0, _OCHUNK // (_L * 8), gbody, 0, unroll=2)
                pend[s] = pltpu.async_copy(
                    out_r.at[s],
                    eT.at[ev_base + c, pl.ds(h * _OCHUNK, _OCHUNK)], semO)
            if k < _RPW - 1:
                dR = pltpu.async_copy(tbl.at[c + 1], row_v, semR)
        for s in range(2):
            if pend[s] is not None:
                pend[s].wait()

    @pl.when(wid < 8)
    def _():
        do_table(t0T, 0, i0, wid * _RPW)

    @pl.when((wid >= 8) & (wid < 16))
    def _():
        do_table(t1T, D0, i1, (wid - 8) * _RPW)

    @pl.when(wid >= 16)
    def _():
        do_table(t2T, D0 + D1, i2, (wid - 16) * _RPW)


@functools.cache
def _make_gather():
    return pl.kernel(
        _gather_body,
        out_type=jax.ShapeDtypeStruct((DTOT, BATCH), jnp.float32),
        mesh=plsc.VectorSubcoreMesh(core_axis_name="c", subcore_axis_name="s"),
        scratch_types=[
            pltpu.VMEM((BATCH,), jnp.int32),
            pltpu.VMEM((V,), jnp.float32),
            pltpu.VMEM((2, _OCHUNK), jnp.float32),
            pltpu.SemaphoreType.DMA,
            pltpu.SemaphoreType.DMA,
            pltpu.SemaphoreType.DMA,
        ],
        compiler_params=pltpu.CompilerParams(needs_layout_passes=False),
    )


_MM_COLS = 4096


def _mm_body(e_ref, w_ref, b_ref, o_ref):
    dn = (((0,), (0,)), ((), ()))
    acc = lax.dot_general(e_ref[...], w_ref[...], dn,
                          preferred_element_type=jnp.float32)
    o_ref[...] = acc + b_ref[...]


_matmul = pl.pallas_call(
    _mm_body,
    grid=(BATCH // _MM_COLS,),
    in_specs=[
        pl.BlockSpec((DTOT, _MM_COLS), lambda i: (0, i)),
        pl.BlockSpec((DTOT, HIDDEN), lambda i: (0, 0)),
        pl.BlockSpec((1, HIDDEN), lambda i: (0, 0)),
    ],
    out_specs=pl.BlockSpec((_MM_COLS, HIDDEN), lambda i: (i, 0)),
    out_shape=jax.ShapeDtypeStruct((BATCH, HIDDEN), jnp.float32),
)


@jax.jit
def kernel(segment_features, lane_table, type_table, length_table, W, b):
    idx = segment_features.astype(jnp.int32)
    eT = _make_gather()(
        idx[:, 0], idx[:, 1], idx[:, 2],
        lane_table.T, type_table.T, length_table.T)
    return _matmul(eT, W.T, b.reshape(1, HIDDEN))


# speedup vs baseline: 2.2539x; 1.0028x over previous
import functools

import jax
import jax.numpy as jnp
from jax import lax
from jax.experimental import pallas as pl
from jax.experimental.pallas import tpu as pltpu
from jax.experimental.pallas import tpu_sc as plsc

BATCH = 16384
D0 = 32
D1 = 32
D2 = 64
DTOT = D0 + D1 + D2
HIDDEN = 128
V = 100000

_NC = 2
_NS = 16
_NW = _NC * _NS
_RPW = DTOT // _NW
_OCHUNK = 4096
_L = 16


def _gather_body(segT, t0T, t1T, t2T, eT,
                 idx_v, row_v, out_r, semI, semR, semO):
    wid = lax.axis_index("s") * _NC + lax.axis_index("c")

    @pl.when((wid & 3) > 0)
    def _():
        t = lax.fori_loop(0, 750 * (wid & 3), lambda i, a: a + 1, 0)
        out_r[0, pl.ds(0, _L)] = jnp.full((_L,), t, jnp.float32)

    def do_table(tbl, ev_base, ti, base):
        dI = pltpu.async_copy(segT.at[pl.ds(ti, 1)], idx_v, semI)
        dR = pltpu.async_copy(tbl.at[base], row_v, semR)
        dI.wait()
        pend = [None, None]
        for k in range(_RPW):
            c = base + k
            dR.wait()
            for h in range(BATCH // _OCHUNK):
                s = h % 2
                if pend[s] is not None:
                    pend[s].wait()
                    pend[s] = None

                def gbody(j, carry, h=h, s=s):
                    off = j * (_L * 8)
                    ivs = [idx_v[0, pl.ds(h * _OCHUNK + off + t * _L, _L)]
                           for t in range(8)]
                    gs = [plsc.load_gather(row_v, [iv]) for iv in ivs]
                    for t in range(8):
                        out_r[s, pl.ds(off + t * _L, _L)] = gs[t]
                    return carry
                lax.fori_loop(0, _OCHUNK // (_L * 8), gbody, 0, unroll=2)
                pend[s] = pltpu.async_copy(
                    out_r.at[s],
                    eT.at[ev_base + c, pl.ds(h * _OCHUNK, _OCHUNK)], semO)
            if k < _RPW - 1:
                dR = pltpu.async_copy(tbl.at[c + 1], row_v, semR)
        for s in range(2):
            if pend[s] is not None:
                pend[s].wait()

    @pl.when(wid < 8)
    def _():
        do_table(t0T, 0, 0, wid * _RPW)

    @pl.when((wid >= 8) & (wid < 16))
    def _():
        do_table(t1T, D0, 1, (wid - 8) * _RPW)

    @pl.when(wid >= 16)
    def _():
        do_table(t2T, D0 + D1, 2, (wid - 16) * _RPW)


@functools.cache
def _make_gather():
    return pl.kernel(
        _gather_body,
        out_type=jax.ShapeDtypeStruct((DTOT, BATCH), jnp.float32),
        mesh=plsc.VectorSubcoreMesh(core_axis_name="c", subcore_axis_name="s"),
        scratch_types=[
            pltpu.VMEM((1, BATCH), jnp.int32),
            pltpu.VMEM((V,), jnp.float32),
            pltpu.VMEM((2, _OCHUNK), jnp.float32),
            pltpu.SemaphoreType.DMA,
            pltpu.SemaphoreType.DMA,
            pltpu.SemaphoreType.DMA,
        ],
        compiler_params=pltpu.CompilerParams(needs_layout_passes=False),
    )


_MM_COLS = 4096


def _mm_body(e_ref, w_ref, b_ref, o_ref):
    dn = (((0,), (0,)), ((), ()))
    acc = lax.dot_general(e_ref[...], w_ref[...], dn,
                          preferred_element_type=jnp.float32)
    o_ref[...] = acc + b_ref[...]


_matmul = pl.pallas_call(
    _mm_body,
    grid=(BATCH // _MM_COLS,),
    in_specs=[
        pl.BlockSpec((DTOT, _MM_COLS), lambda i: (0, i)),
        pl.BlockSpec((DTOT, HIDDEN), lambda i: (0, 0)),
        pl.BlockSpec((1, HIDDEN), lambda i: (0, 0)),
    ],
    out_specs=pl.BlockSpec((_MM_COLS, HIDDEN), lambda i: (i, 0)),
    out_shape=jax.ShapeDtypeStruct((BATCH, HIDDEN), jnp.float32),
)


@jax.jit
def kernel(segment_features, lane_table, type_table, length_table, W, b):
    eT = _make_gather()(
        segment_features.astype(jnp.int32).T,
        lane_table.T, type_table.T, length_table.T)
    return _matmul(eT, W.T, b.reshape(1, HIDDEN))


# speedup vs baseline: 2.3510x; 1.0431x over previous
import functools

import jax
import jax.numpy as jnp
from jax import lax
from jax.experimental import pallas as pl
from jax.experimental.pallas import tpu as pltpu
from jax.experimental.pallas import tpu_sc as plsc

BATCH = 16384
D0 = 32
D1 = 32
D2 = 64
DTOT = D0 + D1 + D2
HIDDEN = 128
V = 100000

_NC = 2
_NS = 16
_NW = _NC * _NS
_RPW = DTOT // _NW
_OCHUNK = 4096
_L = 16


def _gather_body(segT, t0T, t1T, t2T, eT,
                 idx_v, row_v, out_r, semI, semR, semO):
    wid = lax.axis_index("s") * _NC + lax.axis_index("c")

    @pl.when((wid & 3) > 0)
    def _():
        t = lax.fori_loop(0, 750 * (wid & 3), lambda i, a: a + 1, 0)
        out_r[0, pl.ds(0, _L)] = jnp.full((_L,), t, jnp.float32)

    def do_table(tbl, ev_base, ti, base):
        dI = pltpu.async_copy(segT.at[pl.ds(ti, 1)], idx_v, semI)
        dR = pltpu.async_copy(tbl.at[base], row_v, semR)
        dI.wait()
        pend = [None, None]
        for k in range(_RPW):
            c = base + k
            dR.wait()
            for h in range(BATCH // _OCHUNK):
                s = h % 2
                if pend[s] is not None:
                    pend[s].wait()
                    pend[s] = None

                def gbody(j, carry, h=h, s=s):
                    off = j * (_L * 8)
                    ivs = [idx_v[0, pl.ds(h * _OCHUNK + off + t * _L, _L)]
                           for t in range(8)]
                    gs = [plsc.load_gather(row_v, [iv]) for iv in ivs]
                    for t in range(8):
                        out_r[s, pl.ds(off + t * _L, _L)] = gs[t]
                    return carry
                lax.fori_loop(0, _OCHUNK // (_L * 8), gbody, 0, unroll=1)
                pend[s] = pltpu.async_copy(
                    out_r.at[s],
                    eT.at[ev_base + c, pl.ds(h * _OCHUNK, _OCHUNK)], semO)
            if k < _RPW - 1:
                dR = pltpu.async_copy(tbl.at[c + 1], row_v, semR)
        for s in range(2):
            if pend[s] is not None:
                pend[s].wait()

    @pl.when(wid < 8)
    def _():
        do_table(t0T, 0, 0, wid * _RPW)

    @pl.when((wid >= 8) & (wid < 16))
    def _():
        do_table(t1T, D0, 1, (wid - 8) * _RPW)

    @pl.when(wid >= 16)
    def _():
        do_table(t2T, D0 + D1, 2, (wid - 16) * _RPW)


@functools.cache
def _make_gather():
    return pl.kernel(
        _gather_body,
        out_type=jax.ShapeDtypeStruct((DTOT, BATCH), jnp.float32),
        mesh=plsc.VectorSubcoreMesh(core_axis_name="c", subcore_axis_name="s"),
        scratch_types=[
            pltpu.VMEM((1, BATCH), jnp.int32),
            pltpu.VMEM((V,), jnp.float32),
            pltpu.VMEM((2, _OCHUNK), jnp.float32),
            pltpu.SemaphoreType.DMA,
            pltpu.SemaphoreType.DMA,
            pltpu.SemaphoreType.DMA,
        ],
        compiler_params=pltpu.CompilerParams(needs_layout_passes=False),
    )


_MM_COLS = 4096


def _mm_body(e_ref, w_ref, b_ref, o_ref):
    dn = (((0,), (0,)), ((), ()))
    acc = lax.dot_general(e_ref[...], w_ref[...], dn,
                          preferred_element_type=jnp.float32)
    o_ref[...] = acc + b_ref[...]


_matmul = pl.pallas_call(
    _mm_body,
    grid=(BATCH // _MM_COLS,),
    in_specs=[
        pl.BlockSpec((DTOT, _MM_COLS), lambda i: (0, i)),
        pl.BlockSpec((DTOT, HIDDEN), lambda i: (0, 0)),
        pl.BlockSpec((1, HIDDEN), lambda i: (0, 0)),
    ],
    out_specs=pl.BlockSpec((_MM_COLS, HIDDEN), lambda i: (i, 0)),
    out_shape=jax.ShapeDtypeStruct((BATCH, HIDDEN), jnp.float32),
)


@jax.jit
def kernel(segment_features, lane_table, type_table, length_table, W, b):
    eT = _make_gather()(
        segment_features.astype(jnp.int32).T,
        lane_table.T, type_table.T, length_table.T)
    return _matmul(eT, W.T, b.reshape(1, HIDDEN))


# speedup vs baseline: 2.5676x; 1.0921x over previous
import functools

import jax
import jax.numpy as jnp
from jax import lax
from jax.experimental import pallas as pl
from jax.experimental.pallas import tpu as pltpu
from jax.experimental.pallas import tpu_sc as plsc

BATCH = 16384
D0 = 32
D1 = 32
D2 = 64
DTOT = D0 + D1 + D2
HIDDEN = 128
V = 100000

_NC = 2
_NS = 16
_NW = _NC * _NS
_RPW = DTOT // _NW
_OCHUNK = 4096
_L = 16


def _gather_body(segT, t0T, t1T, t2T, eT,
                 idx_v, row_v, out_r, semI, semR, semO):
    wid = lax.axis_index("s") * _NC + lax.axis_index("c")

    @pl.when((wid & 3) > 0)
    def _():
        t = lax.fori_loop(0, 750 * (wid & 3), lambda i, a: a + 1, 0)
        out_r[0, pl.ds(0, _L)] = jnp.full((_L,), t, jnp.float32)

    def do_table(tbl, ev_base, ti, base):
        dI = pltpu.async_copy(segT.at[pl.ds(ti, 1)], idx_v, semI)
        pltpu.async_copy(tbl.at[base], row_v, semR)
        dI.wait()

        def row_body(k, carry):
            c = base + k
            pltpu.make_async_copy(tbl.at[base], row_v, semR).wait()
            for h in range(BATCH // _OCHUNK):
                s = h % 2

                @pl.when((k > 0) | (h >= 2))
                def _():
                    pltpu.make_async_copy(
                        out_r.at[s], eT.at[ev_base, pl.ds(0, _OCHUNK)],
                        semO).wait()

                def gbody(j, carry, h=h, s=s):
                    off = j * (_L * 8)
                    ivs = [idx_v[0, pl.ds(h * _OCHUNK + off + t * _L, _L)]
                           for t in range(8)]
                    gs = [plsc.load_gather(row_v, [iv]) for iv in ivs]
                    for t in range(8):
                        out_r[s, pl.ds(off + t * _L, _L)] = gs[t]
                    return carry
                lax.fori_loop(0, _OCHUNK // (_L * 8), gbody, 0, unroll=1)
                pltpu.async_copy(
                    out_r.at[s],
                    eT.at[ev_base + c, pl.ds(h * _OCHUNK, _OCHUNK)], semO)

            @pl.when(k < _RPW - 1)
            def _():
                pltpu.async_copy(tbl.at[c + 1], row_v, semR)
            return carry

        lax.fori_loop(0, _RPW, row_body, 0, unroll=1)
        for _ in range(2):
            pltpu.make_async_copy(
                out_r.at[0], eT.at[ev_base, pl.ds(0, _OCHUNK)], semO).wait()

    @pl.when(wid < 8)
    def _():
        do_table(t0T, 0, 0, wid * _RPW)

    @pl.when((wid >= 8) & (wid < 16))
    def _():
        do_table(t1T, D0, 1, (wid - 8) * _RPW)

    @pl.when(wid >= 16)
    def _():
        do_table(t2T, D0 + D1, 2, (wid - 16) * _RPW)


@functools.cache
def _make_gather():
    return pl.kernel(
        _gather_body,
        out_type=jax.ShapeDtypeStruct((DTOT, BATCH), jnp.float32),
        mesh=plsc.VectorSubcoreMesh(core_axis_name="c", subcore_axis_name="s"),
        scratch_types=[
            pltpu.VMEM((1, BATCH), jnp.int32),
            pltpu.VMEM((V,), jnp.float32),
            pltpu.VMEM((2, _OCHUNK), jnp.float32),
            pltpu.SemaphoreType.DMA,
            pltpu.SemaphoreType.DMA,
            pltpu.SemaphoreType.DMA,
        ],
        compiler_params=pltpu.CompilerParams(needs_layout_passes=False),
    )


_MM_COLS = 4096


def _mm_body(e_ref, w_ref, b_ref, o_ref):
    dn = (((0,), (0,)), ((), ()))
    acc = lax.dot_general(e_ref[...], w_ref[...], dn,
                          preferred_element_type=jnp.float32)
    o_ref[...] = acc + b_ref[...]


_matmul = pl.pallas_call(
    _mm_body,
    grid=(BATCH // _MM_COLS,),
    in_specs=[
        pl.BlockSpec((DTOT, _MM_COLS), lambda i: (0, i)),
        pl.BlockSpec((DTOT, HIDDEN), lambda i: (0, 0)),
        pl.BlockSpec((1, HIDDEN), lambda i: (0, 0)),
    ],
    out_specs=pl.BlockSpec((_MM_COLS, HIDDEN), lambda i: (i, 0)),
    out_shape=jax.ShapeDtypeStruct((BATCH, HIDDEN), jnp.float32),
)


@jax.jit
def kernel(segment_features, lane_table, type_table, length_table, W, b):
    eT = _make_gather()(
        segment_features.astype(jnp.int32).T,
        lane_table.T, type_table.T, length_table.T)
    return _matmul(eT, W.T, b.reshape(1, HIDDEN))


# speedup vs baseline: 2.7067x; 1.0542x over previous
import functools

import jax
import jax.numpy as jnp
from jax import lax
from jax.experimental import pallas as pl
from jax.experimental.pallas import tpu as pltpu
from jax.experimental.pallas import tpu_sc as plsc

BATCH = 16384
D0 = 32
D1 = 32
D2 = 64
DTOT = D0 + D1 + D2
HIDDEN = 128
V = 100000

_NC = 2
_NS = 16
_NW = _NC * _NS
_RPW = DTOT // _NW
_OCHUNK = 4096
_L = 16


def _gather_body(segT, t0T, t1T, t2T, eT,
                 idx_v, row_v, out_r, semI, semR, semO):
    wid = lax.axis_index("s") * _NC + lax.axis_index("c")

    @pl.when((wid & 3) > 0)
    def _():
        t = lax.fori_loop(0, 750 * (wid & 3), lambda i, a: a + 1, 0)
        out_r[pl.ds(0, _L)] = jnp.full((_L,), t, jnp.float32)

    def do_table(tbl, ev_base, ti, base):
        dI = pltpu.async_copy(segT.at[pl.ds(ti, 1)], idx_v, semI)
        pltpu.async_copy(tbl.at[base], row_v, semR)
        dI.wait()

        def row_body(k, carry):
            c = base + k
            pltpu.make_async_copy(tbl.at[base], row_v, semR).wait()

            def chunk_body(h, carry2):
                so = (h % 2) * _OCHUNK

                @pl.when((k > 0) | (h >= 2))
                def _():
                    pltpu.make_async_copy(
                        out_r.at[pl.ds(0, _OCHUNK)],
                        eT.at[ev_base, pl.ds(0, _OCHUNK)], semO).wait()

                def gbody(j, carry):
                    off = j * (_L * 8)
                    ivs = [idx_v[0, pl.ds(h * _OCHUNK + off + t * _L, _L)]
                           for t in range(8)]
                    gs = [plsc.load_gather(row_v, [iv]) for iv in ivs]
                    for t in range(8):
                        out_r[pl.ds(so + off + t * _L, _L)] = gs[t]
                    return carry
                lax.fori_loop(0, _OCHUNK // (_L * 8), gbody, 0, unroll=1)
                pltpu.async_copy(
                    out_r.at[pl.ds(so, _OCHUNK)],
                    eT.at[ev_base + c, pl.ds(h * _OCHUNK, _OCHUNK)], semO)
                return carry2

            lax.fori_loop(0, BATCH // _OCHUNK, chunk_body, 0, unroll=1)

            @pl.when(k < _RPW - 1)
            def _():
                pltpu.async_copy(tbl.at[c + 1], row_v, semR)
            return carry

        lax.fori_loop(0, _RPW, row_body, 0, unroll=1)
        for _ in range(2):
            pltpu.make_async_copy(
                out_r.at[pl.ds(0, _OCHUNK)],
                eT.at[ev_base, pl.ds(0, _OCHUNK)], semO).wait()

    @pl.when(wid < 8)
    def _():
        do_table(t0T, 0, 0, wid * _RPW)

    @pl.when((wid >= 8) & (wid < 16))
    def _():
        do_table(t1T, D0, 1, (wid - 8) * _RPW)

    @pl.when(wid >= 16)
    def _():
        do_table(t2T, D0 + D1, 2, (wid - 16) * _RPW)


@functools.cache
def _make_gather():
    return pl.kernel(
        _gather_body,
        out_type=jax.ShapeDtypeStruct((DTOT, BATCH), jnp.float32),
        mesh=plsc.VectorSubcoreMesh(core_axis_name="c", subcore_axis_name="s"),
        scratch_types=[
            pltpu.VMEM((1, BATCH), jnp.int32),
            pltpu.VMEM((V,), jnp.float32),
            pltpu.VMEM((2 * _OCHUNK,), jnp.float32),
            pltpu.SemaphoreType.DMA,
            pltpu.SemaphoreType.DMA,
            pltpu.SemaphoreType.DMA,
        ],
        compiler_params=pltpu.CompilerParams(needs_layout_passes=False),
    )


_MM_COLS = 4096


def _mm_body(e_ref, w_ref, b_ref, o_ref):
    dn = (((0,), (0,)), ((), ()))
    acc = lax.dot_general(e_ref[...], w_ref[...], dn,
                          preferred_element_type=jnp.float32)
    o_ref[...] = acc + b_ref[...]


_matmul = pl.pallas_call(
    _mm_body,
    grid=(BATCH // _MM_COLS,),
    in_specs=[
        pl.BlockSpec((DTOT, _MM_COLS), lambda i: (0, i)),
        pl.BlockSpec((DTOT, HIDDEN), lambda i: (0, 0)),
        pl.BlockSpec((1, HIDDEN), lambda i: (0, 0)),
    ],
    out_specs=pl.BlockSpec((_MM_COLS, HIDDEN), lambda i: (i, 0)),
    out_shape=jax.ShapeDtypeStruct((BATCH, HIDDEN), jnp.float32),
)


@jax.jit
def kernel(segment_features, lane_table, type_table, length_table, W, b):
    eT = _make_gather()(
        segment_features.astype(jnp.int32).T,
        lane_table.T, type_table.T, length_table.T)
    return _matmul(eT, W.T, b.reshape(1, HIDDEN))
